# baseline (speedup 1.0000x reference)
import functools

import jax
import jax.numpy as jnp
from jax import lax
from jax.experimental import pallas as pl
from jax.experimental.pallas import tpu as pltpu
from jax.experimental.pallas import tpu_sc as plsc

N = 10000
E = 320000
D = 128
DH = D // 2
NP = 10240
NC = 2
NS = 16
NW = NC * NS
EW = E // NW
C = 125
CH = EW // C
RPS = NP // NS
NRING = 5

_MESH = plsc.VectorSubcoreMesh(core_axis_name="c", subcore_axis_name="s")


def _fill_f32(ref, rows, cols, value):
    groups = cols // 16
    vec = jnp.full((16,), value, jnp.float32)

    def body(k, _):
        r = k // groups
        j = k % groups
        ref[r, pl.ds(j * 16, 16)] = vec
        return 0

    lax.fori_loop(0, rows * groups, body, 0)


def _fill_f32_1d(ref, n, value):
    vec = jnp.full((16,), value, jnp.float32)

    def body(k, _):
        ref[pl.ds(k * 16, 16)] = vec
        return 0

    lax.fori_loop(0, n // 16, body, 0)


@functools.partial(
    pl.kernel,
    out_type=jax.ShapeDtypeStruct((NC, NP), jnp.float32),
    mesh=_MESH,
    scratch_types=[
        pltpu.VMEM((CH, C), jnp.int32),
        pltpu.VMEM((128,), jnp.float32),
        pltpu.VMEM((RPS,), jnp.float32),
        pltpu.VMEM_SHARED((NP,), jnp.float32),
    ],
    compiler_params=pltpu.CompilerParams(use_tc_tiling_on_sc=False),
)
def _deg_kernel(ei_hbm, out_hbm, idx_v, ones_v, z_v, cnt_sh):
    cid = lax.axis_index("c")
    sid = lax.axis_index("s")
    wid = sid * NC + cid

    _fill_f32_1d(ones_v, 128, 1.0)
    _fill_f32_1d(z_v, RPS, 0.0)

    pltpu.sync_copy(z_v, cnt_sh.at[pl.ds(sid * RPS, RPS)])
    pltpu.sync_copy(ei_hbm.at[1, wid], idx_v)
    plsc.subcore_barrier()

    def body(ci, _):
        pltpu.sync_copy(ones_v.at[pl.ds(0, C)], cnt_sh.at[idx_v.at[ci]],
                        add=True)
        return 0

    lax.fori_loop(0, CH, body, 0)
    plsc.subcore_barrier()
    pltpu.sync_copy(cnt_sh.at[pl.ds(sid * RPS, RPS)],
                    out_hbm.at[cid, pl.ds(sid * RPS, RPS)])


def _matmul_body(x_ref, w_ref, h_ref):
    h_ref[...] = jnp.dot(x_ref[...], w_ref[...],
                         preferred_element_type=jnp.float32)


def _matmul(x, w):
    blk = NP // 8
    return pl.pallas_call(
        _matmul_body,
        grid=(8,),
        in_specs=[
            pl.BlockSpec((blk, D), lambda i: (i, 0)),
            pl.BlockSpec((D, D), lambda i: (0, 0)),
        ],
        out_specs=pl.BlockSpec((blk, D), lambda i: (i, 0)),
        out_shape=jax.ShapeDtypeStruct((NP, D), jnp.float32),
    )(x, w)


def _scale_body(cnt_ref, h_ref, ga_ref, gb_ref):
    deg = cnt_ref[0, :] + cnt_ref[1, :] + 1.0
    dis = lax.rsqrt(deg)
    g = h_ref[...] * dis[:, None]
    ga_ref[...] = g[:, :DH]
    gb_ref[...] = g[:, DH:]


def _transform(counts, h):
    blk = NP // 8
    return pl.pallas_call(
        _scale_body,
        grid=(8,),
        in_specs=[
            pl.BlockSpec((NC, blk), lambda i: (0, i)),
            pl.BlockSpec((blk, D), lambda i: (i, 0)),
        ],
        out_specs=[
            pl.BlockSpec((blk, DH), lambda i: (i, 0)),
            pl.BlockSpec((blk, DH), lambda i: (i, 0)),
        ],
        out_shape=[
            jax.ShapeDtypeStruct((NP, DH), jnp.float32),
            jax.ShapeDtypeStruct((NP, DH), jnp.float32),
        ],
    )(counts, h)


@functools.partial(
    pl.kernel,
    out_type=jax.ShapeDtypeStruct((NC, NP, DH), jnp.float32),
    mesh=_MESH,
    scratch_types=[
        pltpu.VMEM((CH, C), jnp.int32),
        pltpu.VMEM((CH, C), jnp.int32),
        pltpu.VMEM((C, DH), jnp.float32),
        pltpu.VMEM((C, DH), jnp.float32),
        pltpu.VMEM((C, DH), jnp.float32),
        pltpu.VMEM((C, DH), jnp.float32),
        pltpu.VMEM((C, DH), jnp.float32),
        pltpu.VMEM_SHARED((NP, DH), jnp.float32),
        pltpu.SemaphoreType.DMA,
        pltpu.SemaphoreType.DMA,
        pltpu.SemaphoreType.DMA,
        pltpu.SemaphoreType.DMA,
        pltpu.SemaphoreType.DMA,
    ],
    compiler_params=pltpu.CompilerParams(use_tc_tiling_on_sc=False),
)
def _agg_kernel(ei_hbm, g_hbm, outv_hbm,
                src_v, dst_v, rows0, rows1, rows2, rows3, rows4, acc_sh,
                sem0, sem1, sem2, sem3, sem4):
    cid = lax.axis_index("c")
    sid = lax.axis_index("s")
    wid = sid * NC + cid
    rings = ((rows0, sem0), (rows1, sem1), (rows2, sem2), (rows3, sem3),
             (rows4, sem4))

    pltpu.sync_copy(ei_hbm.at[0, wid], src_v)
    pltpu.sync_copy(ei_hbm.at[1, wid], dst_v)

    @pl.when(cid == 0)
    def _():
        pltpu.sync_copy(g_hbm.at[pl.ds(sid * RPS, RPS)],
                        acc_sh.at[pl.ds(sid * RPS, RPS)])

    @pl.when(cid != 0)
    def _():
        _fill_f32(rows0, C, DH, 0.0)
        for i in range(RPS // C):
            pltpu.sync_copy(rows0, acc_sh.at[pl.ds(sid * RPS + i * C, C)])
        pltpu.sync_copy(
            rows0.at[pl.ds(0, RPS % C)],
            acc_sh.at[pl.ds(sid * RPS + (RPS // C) * C, RPS % C)])

    plsc.subcore_barrier()

    for s, (rows, sem) in enumerate(rings):
        pltpu.async_copy(g_hbm.at[src_v.at[s]], rows, sem)

    def step(c, rows, sem):
        pltpu.make_async_copy(g_hbm.at[src_v.at[c]], rows, sem).wait()
        pltpu.sync_copy(rows, acc_sh.at[dst_v.at[c]], add=True)

        @pl.when(c + NRING < CH)
        def _():
            pltpu.async_copy(g_hbm.at[src_v.at[c + NRING]], rows, sem)

    def body(i, _):
        for s, (rows, sem) in enumerate(rings):
            step(NRING * i + s, rows, sem)
        return 0

    lax.fori_loop(0, CH // NRING, body, 0)
    plsc.subcore_barrier()
    pltpu.sync_copy(acc_sh.at[pl.ds(sid * RPS, RPS)],
                    outv_hbm.at[cid, pl.ds(sid * RPS, RPS)])


def _final_body(cnt_ref, acca_ref, accb_ref, b_ref, o_ref):
    deg = cnt_ref[0, :] + cnt_ref[1, :] + 1.0
    dis = lax.rsqrt(deg)
    sa = acca_ref[0] + acca_ref[1]
    sb = accb_ref[0] + accb_ref[1]
    s = jnp.concatenate([sa, sb], axis=1)
    o_ref[...] = jnp.maximum(s * dis[:, None] + b_ref[...], 0.0)


def _finalize(counts, acca, accb, b2):
    blk = NP // 8
    return pl.pallas_call(
        _final_body,
        grid=(8,),
        in_specs=[
            pl.BlockSpec((NC, blk), lambda i: (0, i)),
            pl.BlockSpec((NC, blk, DH), lambda i: (0, i, 0)),
            pl.BlockSpec((NC, blk, DH), lambda i: (0, i, 0)),
            pl.BlockSpec((1, D), lambda i: (0, 0)),
        ],
        out_specs=pl.BlockSpec((blk, D), lambda i: (i, 0)),
        out_shape=jax.ShapeDtypeStruct((N, D), jnp.float32),
    )(counts, acca, accb, b2)


def kernel(x, edge_index, W, b):
    ei = edge_index.astype(jnp.int32).reshape(2, NW, CH, C)
    h = _matmul(x, W)
    counts = _deg_kernel(ei)
    ga, gb = _transform(counts, h)
    acca = _agg_kernel(ei, ga)
    accb = _agg_kernel(ei, gb)
    return _finalize(counts, acca, accb, b.reshape(1, D))


# speedup vs baseline: 1.0056x; 1.0056x over previous
import functools

import jax
import jax.numpy as jnp
from jax import lax
from jax.experimental import pallas as pl
from jax.experimental.pallas import tpu as pltpu
from jax.experimental.pallas import tpu_sc as plsc

N = 10000
E = 320000
D = 128
DH = D // 2
NP = 10240
NC = 2
NS = 16
NW = NC * NS
EW = E // NW
C = 125
CH = EW // C
RPS = NP // NS
NRING = 5

_MESH = plsc.VectorSubcoreMesh(core_axis_name="c", subcore_axis_name="s")


def _fill_f32(ref, rows, cols, value):
    groups = cols // 16
    vec = jnp.full((16,), value, jnp.float32)

    def body(k, _):
        r = k // groups
        j = k % groups
        ref[r, pl.ds(j * 16, 16)] = vec
        return 0

    lax.fori_loop(0, rows * groups, body, 0)


def _fill_f32_1d(ref, n, value):
    vec = jnp.full((16,), value, jnp.float32)

    def body(k, _):
        ref[pl.ds(k * 16, 16)] = vec
        return 0

    lax.fori_loop(0, n // 16, body, 0)


@functools.partial(
    pl.kernel,
    out_type=jax.ShapeDtypeStruct((NC, NP), jnp.float32),
    mesh=_MESH,
    scratch_types=[
        pltpu.VMEM((CH, C), jnp.int32),
        pltpu.VMEM((128,), jnp.float32),
        pltpu.VMEM((RPS,), jnp.float32),
        pltpu.VMEM_SHARED((NP,), jnp.float32),
    ],
    compiler_params=pltpu.CompilerParams(use_tc_tiling_on_sc=False),
)
def _deg_kernel(ei_hbm, out_hbm, idx_v, ones_v, z_v, cnt_sh):
    cid = lax.axis_index("c")
    sid = lax.axis_index("s")
    wid = sid * NC + cid

    _fill_f32_1d(ones_v, 128, 1.0)
    _fill_f32_1d(z_v, RPS, 0.0)

    pltpu.sync_copy(z_v, cnt_sh.at[pl.ds(sid * RPS, RPS)])
    pltpu.sync_copy(ei_hbm.at[1, wid], idx_v)
    plsc.subcore_barrier()

    def body(ci, _):
        pltpu.sync_copy(ones_v.at[pl.ds(0, C)], cnt_sh.at[idx_v.at[ci]],
                        add=True)
        return 0

    lax.fori_loop(0, CH, body, 0)
    plsc.subcore_barrier()
    pltpu.sync_copy(cnt_sh.at[pl.ds(sid * RPS, RPS)],
                    out_hbm.at[cid, pl.ds(sid * RPS, RPS)])


def _transform_body(cnt_ref, x_ref, w_ref, ga_ref, gb_ref):
    deg = cnt_ref[0, :] + cnt_ref[1, :] + 1.0
    dis = lax.rsqrt(deg)
    h = jnp.dot(x_ref[...], w_ref[...], preferred_element_type=jnp.float32)
    g = h * dis[:, None]
    ga_ref[...] = g[:, :DH]
    gb_ref[...] = g[:, DH:]


def _transform(counts, x, w):
    blk = NP // 8
    return pl.pallas_call(
        _transform_body,
        grid=(8,),
        in_specs=[
            pl.BlockSpec((NC, blk), lambda i: (0, i)),
            pl.BlockSpec((blk, D), lambda i: (i, 0)),
            pl.BlockSpec((D, D), lambda i: (0, 0)),
        ],
        out_specs=[
            pl.BlockSpec((blk, DH), lambda i: (i, 0)),
            pl.BlockSpec((blk, DH), lambda i: (i, 0)),
        ],
        out_shape=[
            jax.ShapeDtypeStruct((NP, DH), jnp.float32),
            jax.ShapeDtypeStruct((NP, DH), jnp.float32),
        ],
    )(counts, x, w)


@functools.partial(
    pl.kernel,
    out_type=jax.ShapeDtypeStruct((NC, NP, DH), jnp.float32),
    mesh=_MESH,
    scratch_types=[
        pltpu.VMEM((CH, C), jnp.int32),
        pltpu.VMEM((CH, C), jnp.int32),
        pltpu.VMEM((C, DH), jnp.float32),
        pltpu.VMEM((C, DH), jnp.float32),
        pltpu.VMEM((C, DH), jnp.float32),
        pltpu.VMEM((C, DH), jnp.float32),
        pltpu.VMEM((C, DH), jnp.float32),
        pltpu.VMEM_SHARED((NP, DH), jnp.float32),
        pltpu.SemaphoreType.DMA,
        pltpu.SemaphoreType.DMA,
        pltpu.SemaphoreType.DMA,
        pltpu.SemaphoreType.DMA,
        pltpu.SemaphoreType.DMA,
    ],
    compiler_params=pltpu.CompilerParams(use_tc_tiling_on_sc=False),
)
def _agg_kernel(ei_hbm, g_hbm, outv_hbm,
                src_v, dst_v, rows0, rows1, rows2, rows3, rows4, acc_sh,
                sem0, sem1, sem2, sem3, sem4):
    cid = lax.axis_index("c")
    sid = lax.axis_index("s")
    wid = sid * NC + cid
    rings = ((rows0, sem0), (rows1, sem1), (rows2, sem2), (rows3, sem3),
             (rows4, sem4))

    pltpu.sync_copy(ei_hbm.at[0, wid], src_v)
    pltpu.sync_copy(ei_hbm.at[1, wid], dst_v)

    @pl.when(cid == 0)
    def _():
        pltpu.sync_copy(g_hbm.at[pl.ds(sid * RPS, RPS)],
                        acc_sh.at[pl.ds(sid * RPS, RPS)])

    @pl.when(cid != 0)
    def _():
        _fill_f32(rows0, C, DH, 0.0)
        for i in range(RPS // C):
            pltpu.sync_copy(rows0, acc_sh.at[pl.ds(sid * RPS + i * C, C)])
        pltpu.sync_copy(
            rows0.at[pl.ds(0, RPS % C)],
            acc_sh.at[pl.ds(sid * RPS + (RPS // C) * C, RPS % C)])

    plsc.subcore_barrier()

    for s, (rows, sem) in enumerate(rings):
        pltpu.async_copy(g_hbm.at[src_v.at[s]], rows, sem)

    def step(c, rows, sem):
        pltpu.make_async_copy(g_hbm.at[src_v.at[c]], rows, sem).wait()
        pltpu.sync_copy(rows, acc_sh.at[dst_v.at[c]], add=True)

        @pl.when(c + NRING < CH)
        def _():
            pltpu.async_copy(g_hbm.at[src_v.at[c + NRING]], rows, sem)

    def body(i, _):
        for s, (rows, sem) in enumerate(rings):
            step(NRING * i + s, rows, sem)
        return 0

    lax.fori_loop(0, CH // NRING, body, 0)
    plsc.subcore_barrier()
    pltpu.sync_copy(acc_sh.at[pl.ds(sid * RPS, RPS)],
                    outv_hbm.at[cid, pl.ds(sid * RPS, RPS)])


def _final_body(cnt_ref, acca_ref, accb_ref, b_ref, o_ref):
    deg = cnt_ref[0, :] + cnt_ref[1, :] + 1.0
    dis = lax.rsqrt(deg)
    sa = acca_ref[0] + acca_ref[1]
    sb = accb_ref[0] + accb_ref[1]
    s = jnp.concatenate([sa, sb], axis=1)
    o_ref[...] = jnp.maximum(s * dis[:, None] + b_ref[...], 0.0)


def _finalize(counts, acca, accb, b2):
    blk = NP // 8
    return pl.pallas_call(
        _final_body,
        grid=(8,),
        in_specs=[
            pl.BlockSpec((NC, blk), lambda i: (0, i)),
            pl.BlockSpec((NC, blk, DH), lambda i: (0, i, 0)),
            pl.BlockSpec((NC, blk, DH), lambda i: (0, i, 0)),
            pl.BlockSpec((1, D), lambda i: (0, 0)),
        ],
        out_specs=pl.BlockSpec((blk, D), lambda i: (i, 0)),
        out_shape=jax.ShapeDtypeStruct((N, D), jnp.float32),
    )(counts, acca, accb, b2)


def kernel(x, edge_index, W, b):
    ei = edge_index.astype(jnp.int32).reshape(2, NW, CH, C)
    counts = _deg_kernel(ei)
    ga, gb = _transform(counts, x, W)
    acca = _agg_kernel(ei, ga)
    accb = _agg_kernel(ei, gb)
    return _finalize(counts, acca, accb, b.reshape(1, D))
